# baseline (speedup 1.0000x reference)
import functools

import jax
import jax.numpy as jnp
from jax import lax
from jax.experimental import pallas as pl
from jax.experimental.pallas import tpu as pltpu
from jax.experimental.pallas import tpu_sc as plsc

_N_WORKERS = 32
_CHUNK = 128
_LANES = 16
_PACK = 4
_BLK = 8192
_ROWS = _BLK // _PACK


def _pack_body(x_ref, o_ref):
    x = x_ref[...].astype(jnp.bfloat16)
    d = x.shape[0]
    eye = jnp.eye(d, dtype=jnp.bfloat16)
    acc = None
    for c in range(_PACK):
        xc = x[:, c * _ROWS:(c + 1) * _ROWS]
        ec = jnp.pad(eye, ((0, 0), (c * d, (_PACK - 1 - c) * d)))
        t = lax.dot_general(xc, ec, (((0,), (0,)), ((), ())),
                            preferred_element_type=jnp.float32)
        acc = t if acc is None else acc + t
    o_ref[...] = acc


def _pack_table(table_t):
    d, n = table_t.shape
    grid = (n + _BLK - 1) // _BLK
    return pl.pallas_call(
        _pack_body,
        grid=(grid,),
        in_specs=[pl.BlockSpec((d, _BLK), lambda g: (0, g))],
        out_specs=pl.BlockSpec((_ROWS, _PACK * d), lambda g: (g, 0)),
        out_shape=jax.ShapeDtypeStruct((grid * _ROWS, _PACK * d), jnp.float32),
        compiler_params=pltpu.CompilerParams(
            dimension_semantics=("arbitrary",)),
    )(table_t)


@functools.partial(jax.jit, static_argnums=(5, 6))
def _mf_call(idx_flat, uf_p, if_p, user_biases, item_biases, b_per_w, d):
    batch = idx_flat.shape[0] // 2
    n_chunks = b_per_w // _CHUNK
    n_groups_half = (b_per_w // 2) // _LANES
    mesh = plsc.VectorSubcoreMesh(core_axis_name="c", subcore_axis_name="s")

    @functools.partial(
        pl.kernel,
        out_type=jax.ShapeDtypeStruct((batch,), jnp.float32),
        mesh=mesh,
        compiler_params=pltpu.CompilerParams(
            needs_layout_passes=False, use_tc_tiling_on_sc=True),
        scratch_types=[
            pltpu.VMEM((b_per_w,), jnp.int32),
            pltpu.VMEM((b_per_w,), jnp.int32),
            pltpu.VMEM((b_per_w,), jnp.int32),
            pltpu.VMEM((b_per_w,), jnp.int32),
            pltpu.VMEM((b_per_w // 2, _PACK * d), jnp.float32),
            pltpu.VMEM((b_per_w // 2, _PACK * d), jnp.float32),
            pltpu.VMEM((b_per_w,), jnp.float32),
            pltpu.VMEM((b_per_w,), jnp.float32),
            pltpu.VMEM((b_per_w,), jnp.float32),
            pltpu.SemaphoreType.DMA,
            pltpu.SemaphoreType.DMA,
        ],
    )
    def k(idx_hbm, uf_hbm, if_hbm, ub_hbm, ib_hbm, out_hbm,
          u1d, i1d, u2d, i2d, urows_v, irows_v, ub_v, ib_v, out_v,
          sem_b, sem_f):
        wid = lax.axis_index("s") * 2 + lax.axis_index("c")
        base = wid * b_per_w
        pltpu.sync_copy(idx_hbm.at[pl.ds(2 * base, b_per_w)], u1d)
        pltpu.sync_copy(idx_hbm.at[pl.ds(2 * base + b_per_w, b_per_w)], i1d)

        blk_s = _BLK.bit_length() - 1
        row_s = _ROWS.bit_length() - 1

        def packed_row(u):
            return (lax.shift_left(lax.shift_right_logical(u, blk_s), row_s)
                    | (u & (_ROWS - 1)))

        def halve_body(l, _):
            sl = pl.ds(l * _LANES, _LANES)
            u2d[sl] = packed_row(u1d[sl])
            i2d[sl] = packed_row(i1d[sl])
            return 0

        lax.fori_loop(0, b_per_w // _LANES, halve_body, 0)

        bias_copies = []
        for j in range(n_chunks):
            sl = pl.ds(j * _CHUNK, _CHUNK)
            bias_copies.append(pltpu.async_copy(
                ub_hbm.at[u1d.at[sl]], ub_v.at[sl], sem_b))
            bias_copies.append(pltpu.async_copy(
                ib_hbm.at[i1d.at[sl]], ib_v.at[sl], sem_b))

        for h in range(2):
            row_copies = []
            for jj in range(n_chunks // 2):
                j = h * (n_chunks // 2) + jj
                src = pl.ds(j * _CHUNK, _CHUNK)
                dst = pl.ds(jj * _CHUNK, _CHUNK)
                row_copies.append(pltpu.async_copy(
                    uf_hbm.at[u2d.at[src]], urows_v.at[dst], sem_f))
                row_copies.append(pltpu.async_copy(
                    if_hbm.at[i2d.at[src]], irows_v.at[dst], sem_f))
            for c in row_copies:
                c.wait()
            if h == 0:
                for c in bias_copies:
                    c.wait()

            def group_body(g, _):
                p0 = h * (b_per_w // 2) + g * _LANES
                sl = pl.ds(p0, _LANES)
                rows = lax.iota(jnp.int32, _LANES) + g * _LANES
                ucol = (lax.shift_right_logical(u1d[sl], row_s)
                        & (_PACK - 1)) * d
                icol = (lax.shift_right_logical(i1d[sl], row_s)
                        & (_PACK - 1)) * d
                acc = ub_v[sl] + ib_v[sl]
                for dd in range(d):
                    fu = plsc.load_gather(urows_v, [rows, ucol + dd])
                    fv = plsc.load_gather(irows_v, [rows, icol + dd])
                    acc = acc + fu * fv
                out_v[sl] = acc
                return 0

            lax.fori_loop(0, n_groups_half, group_body, 0)

        pltpu.sync_copy(out_v, out_hbm.at[pl.ds(base, b_per_w)])

    return k(idx_flat, uf_p, if_p, user_biases, item_biases)


def kernel(user_item_tuple, user_factors, item_factors, user_biases, item_biases):
    batch = user_item_tuple.shape[0]
    d = user_factors.shape[1]
    b_per_w = batch // _N_WORKERS
    idx_flat = jnp.concatenate(
        [user_item_tuple[:, 0].reshape(_N_WORKERS, b_per_w),
         user_item_tuple[:, 1].reshape(_N_WORKERS, b_per_w)],
        axis=1).reshape(-1)
    uf_p = _pack_table(user_factors.T)
    if_p = _pack_table(item_factors.T)
    return _mf_call(idx_flat, uf_p, if_p,
                    user_biases.reshape(-1), item_biases.reshape(-1),
                    b_per_w, d)


# speedup vs baseline: 1.1995x; 1.1995x over previous
import functools

import jax
import jax.numpy as jnp
import numpy as np
from jax import lax
from jax.experimental import pallas as pl
from jax.experimental.pallas import tpu as pltpu
from jax.experimental.pallas import tpu_sc as plsc

_N_WORKERS = 32
_CHUNK = 128
_LANES = 16
_PACK = 8
_BLK = 8192
_ROWS = _BLK // _PACK


def _sel_const(d, parity):
    e = np.zeros((_PACK * d, _PACK * (d // 2)), np.float32)
    for c in range(_PACK):
        for k in range(d // 2):
            e[c * d + 2 * k + parity, (d // 2) * c + k] = 1.0
    return jnp.asarray(e, dtype=jnp.bfloat16)


def _pack_body(x_ref, ee_ref, eo_ref, o_ref):
    x = x_ref[...].astype(jnp.bfloat16)
    xs = jnp.concatenate(
        [x[:, c * _ROWS:(c + 1) * _ROWS] for c in range(_PACK)], axis=0)
    acc_e = lax.dot_general(xs, ee_ref[...], (((0,), (0,)), ((), ())),
                            preferred_element_type=jnp.float32)
    acc_o = lax.dot_general(xs, eo_ref[...], (((0,), (0,)), ((), ())),
                            preferred_element_type=jnp.float32)
    o_ref[...] = pltpu.pack_elementwise(
        [acc_e, acc_o], packed_dtype=jnp.bfloat16)


def _pack_table(table_t):
    d, n = table_t.shape
    grid = (n + _BLK - 1) // _BLK
    sel_shape = (_PACK * d, _PACK * (d // 2))
    return pl.pallas_call(
        _pack_body,
        grid=(grid,),
        in_specs=[
            pl.BlockSpec((d, _BLK), lambda g: (0, g)),
            pl.BlockSpec(sel_shape, lambda g: (0, 0)),
            pl.BlockSpec(sel_shape, lambda g: (0, 0)),
        ],
        out_specs=pl.BlockSpec((_ROWS, _PACK * (d // 2)), lambda g: (g, 0)),
        out_shape=jax.ShapeDtypeStruct(
            (grid * _ROWS, _PACK * (d // 2)), jnp.int32),
        compiler_params=pltpu.CompilerParams(
            dimension_semantics=("arbitrary",)),
    )(table_t, _sel_const(d, 0), _sel_const(d, 1))


@functools.partial(jax.jit, static_argnums=(5, 6))
def _mf_call(idx_flat, uf_p, if_p, user_biases, item_biases, b_per_w, d2):
    batch = idx_flat.shape[0] // 2
    n_chunks = b_per_w // _CHUNK
    n_groups_half = (b_per_w // 2) // _LANES
    mesh = plsc.VectorSubcoreMesh(core_axis_name="c", subcore_axis_name="s")

    @functools.partial(
        pl.kernel,
        out_type=jax.ShapeDtypeStruct((batch,), jnp.float32),
        mesh=mesh,
        compiler_params=pltpu.CompilerParams(
            needs_layout_passes=False, use_tc_tiling_on_sc=True),
        scratch_types=[
            pltpu.VMEM((b_per_w,), jnp.int32),
            pltpu.VMEM((b_per_w,), jnp.int32),
            pltpu.VMEM((b_per_w,), jnp.int32),
            pltpu.VMEM((b_per_w,), jnp.int32),
            pltpu.VMEM((b_per_w // 2, _PACK * d2), jnp.int32),
            pltpu.VMEM((b_per_w // 2, _PACK * d2), jnp.int32),
            pltpu.VMEM((b_per_w,), jnp.float32),
            pltpu.VMEM((b_per_w,), jnp.float32),
            pltpu.VMEM((b_per_w,), jnp.float32),
            pltpu.SemaphoreType.DMA,
            pltpu.SemaphoreType.DMA,
        ],
    )
    def k(idx_hbm, uf_hbm, if_hbm, ub_hbm, ib_hbm, out_hbm,
          u1d, i1d, u2d, i2d, urows_v, irows_v, ub_v, ib_v, out_v,
          sem_b, sem_f):
        wid = lax.axis_index("s") * 2 + lax.axis_index("c")
        base = wid * b_per_w
        pltpu.sync_copy(idx_hbm.at[pl.ds(2 * base, b_per_w)], u1d)
        pltpu.sync_copy(idx_hbm.at[pl.ds(2 * base + b_per_w, b_per_w)], i1d)

        blk_s = _BLK.bit_length() - 1
        row_s = _ROWS.bit_length() - 1

        def packed_row(u):
            return (lax.shift_left(lax.shift_right_logical(u, blk_s), row_s)
                    | (u & (_ROWS - 1)))

        def halve_body(l, _):
            sl = pl.ds(l * _LANES, _LANES)
            u2d[sl] = packed_row(u1d[sl])
            i2d[sl] = packed_row(i1d[sl])
            return 0

        lax.fori_loop(0, b_per_w // _LANES, halve_body, 0)

        bias_copies = []
        for j in range(n_chunks):
            sl = pl.ds(j * _CHUNK, _CHUNK)
            bias_copies.append(pltpu.async_copy(
                ub_hbm.at[u1d.at[sl]], ub_v.at[sl], sem_b))
            bias_copies.append(pltpu.async_copy(
                ib_hbm.at[i1d.at[sl]], ib_v.at[sl], sem_b))

        for h in range(2):
            row_copies = []
            for jj in range(n_chunks // 2):
                j = h * (n_chunks // 2) + jj
                src = pl.ds(j * _CHUNK, _CHUNK)
                dst = pl.ds(jj * _CHUNK, _CHUNK)
                row_copies.append(pltpu.async_copy(
                    uf_hbm.at[u2d.at[src]], urows_v.at[dst], sem_f))
                row_copies.append(pltpu.async_copy(
                    if_hbm.at[i2d.at[src]], irows_v.at[dst], sem_f))
            for c in row_copies:
                c.wait()
            if h == 0:
                for c in bias_copies:
                    c.wait()

            def group_body(g, _):
                p0 = h * (b_per_w // 2) + g * _LANES
                sl = pl.ds(p0, _LANES)
                rows = lax.iota(jnp.int32, _LANES) + g * _LANES
                ucol = (lax.shift_right_logical(u1d[sl], row_s)
                        & (_PACK - 1)) * d2
                icol = (lax.shift_right_logical(i1d[sl], row_s)
                        & (_PACK - 1)) * d2
                acc = ub_v[sl] + ib_v[sl]
                for dp in range(d2):
                    iu = plsc.load_gather(urows_v, [rows, ucol + dp])
                    iv = plsc.load_gather(irows_v, [rows, icol + dp])
                    ue, uo = plsc.unpack(
                        plsc.bitcast(iu, jnp.bfloat16),
                        format=plsc.PackFormat.INTERLEAVED)
                    ve, vo = plsc.unpack(
                        plsc.bitcast(iv, jnp.bfloat16),
                        format=plsc.PackFormat.INTERLEAVED)
                    acc = acc + ue * ve + uo * vo
                out_v[sl] = acc
                return 0

            lax.fori_loop(0, n_groups_half, group_body, 0)

        pltpu.sync_copy(out_v, out_hbm.at[pl.ds(base, b_per_w)])

    return k(idx_flat, uf_p, if_p, user_biases, item_biases)


def kernel(user_item_tuple, user_factors, item_factors, user_biases, item_biases):
    batch = user_item_tuple.shape[0]
    d = user_factors.shape[1]
    b_per_w = batch // _N_WORKERS
    idx_flat = jnp.concatenate(
        [user_item_tuple[:, 0].reshape(_N_WORKERS, b_per_w),
         user_item_tuple[:, 1].reshape(_N_WORKERS, b_per_w)],
        axis=1).reshape(-1)
    uf_p = _pack_table(user_factors.T)
    if_p = _pack_table(item_factors.T)
    return _mf_call(idx_flat, uf_p, if_p,
                    user_biases.reshape(-1), item_biases.reshape(-1),
                    b_per_w, d // 2)


# speedup vs baseline: 1.4697x; 1.2252x over previous
import functools

import jax
import jax.numpy as jnp
import numpy as np
from jax import lax
from jax.experimental import pallas as pl
from jax.experimental.pallas import tpu as pltpu
from jax.experimental.pallas import tpu_sc as plsc

_N_WORKERS = 32
_CHUNK = 128
_LANES = 16
_PACK = 8
_BLK = 16384
_ROWS = _BLK // _PACK


def _sel_const(d, parity):
    e = np.zeros((_PACK * d, _PACK * (d // 2)), np.float32)
    for c in range(_PACK):
        for k in range(d // 2):
            e[c * d + 2 * k + parity, (d // 2) * c + k] = 1.0
    return jnp.asarray(e, dtype=jnp.bfloat16)


def _pack_body(x_ref, ee_ref, eo_ref, o_ref):
    x = x_ref[...].astype(jnp.bfloat16)
    xs = jnp.concatenate(
        [x[:, c * _ROWS:(c + 1) * _ROWS] for c in range(_PACK)], axis=0)
    acc_e = lax.dot_general(xs, ee_ref[...], (((0,), (0,)), ((), ())),
                            preferred_element_type=jnp.float32)
    acc_o = lax.dot_general(xs, eo_ref[...], (((0,), (0,)), ((), ())),
                            preferred_element_type=jnp.float32)
    o_ref[...] = pltpu.pack_elementwise(
        [acc_e, acc_o], packed_dtype=jnp.bfloat16)


def _pack_table(table_t):
    d, n = table_t.shape
    grid = (n + _BLK - 1) // _BLK
    sel_shape = (_PACK * d, _PACK * (d // 2))
    return pl.pallas_call(
        _pack_body,
        grid=(grid,),
        in_specs=[
            pl.BlockSpec((d, _BLK), lambda g: (0, g)),
            pl.BlockSpec(sel_shape, lambda g: (0, 0)),
            pl.BlockSpec(sel_shape, lambda g: (0, 0)),
        ],
        out_specs=pl.BlockSpec((_ROWS, _PACK * (d // 2)), lambda g: (g, 0)),
        out_shape=jax.ShapeDtypeStruct(
            (grid * _ROWS, _PACK * (d // 2)), jnp.int32),
        compiler_params=pltpu.CompilerParams(
            dimension_semantics=("arbitrary",)),
    )(table_t, _sel_const(d, 0), _sel_const(d, 1))


@functools.partial(jax.jit, static_argnums=(5, 6))
def _mf_call(idx_flat, uf_p, if_p, user_biases, item_biases, b_per_w, d2):
    batch = idx_flat.shape[0] // 2
    n_chunks = b_per_w // _CHUNK
    n_groups_half = (b_per_w // 2) // _LANES
    mesh = plsc.VectorSubcoreMesh(core_axis_name="c", subcore_axis_name="s")

    @functools.partial(
        pl.kernel,
        out_type=jax.ShapeDtypeStruct((batch,), jnp.float32),
        mesh=mesh,
        compiler_params=pltpu.CompilerParams(
            needs_layout_passes=False, use_tc_tiling_on_sc=True),
        scratch_types=[
            pltpu.VMEM((b_per_w,), jnp.int32),
            pltpu.VMEM((b_per_w,), jnp.int32),
            pltpu.VMEM((b_per_w,), jnp.int32),
            pltpu.VMEM((b_per_w,), jnp.int32),
            pltpu.VMEM((b_per_w // 2, _PACK * d2), jnp.int32),
            pltpu.VMEM((b_per_w // 2, _PACK * d2), jnp.int32),
            pltpu.VMEM((b_per_w,), jnp.float32),
            pltpu.VMEM((b_per_w,), jnp.float32),
            pltpu.VMEM((b_per_w,), jnp.float32),
            pltpu.SemaphoreType.DMA,
            pltpu.SemaphoreType.DMA,
        ],
    )
    def k(idx_hbm, uf_hbm, if_hbm, ub_hbm, ib_hbm, out_hbm,
          u1d, i1d, u2d, i2d, urows_v, irows_v, ub_v, ib_v, out_v,
          sem_b, sem_f):
        wid = lax.axis_index("s") * 2 + lax.axis_index("c")
        base = wid * b_per_w
        pltpu.sync_copy(idx_hbm.at[pl.ds(2 * base, b_per_w)], u1d)
        pltpu.sync_copy(idx_hbm.at[pl.ds(2 * base + b_per_w, b_per_w)], i1d)

        blk_s = _BLK.bit_length() - 1
        row_s = _ROWS.bit_length() - 1

        def packed_row(u):
            return (lax.shift_left(lax.shift_right_logical(u, blk_s), row_s)
                    | (u & (_ROWS - 1)))

        def halve_body(l, _):
            sl = pl.ds(l * _LANES, _LANES)
            u2d[sl] = packed_row(u1d[sl])
            i2d[sl] = packed_row(i1d[sl])
            return 0

        lax.fori_loop(0, b_per_w // _LANES, halve_body, 0)

        bias_copies = []
        for j in range(n_chunks):
            sl = pl.ds(j * _CHUNK, _CHUNK)
            bias_copies.append(pltpu.async_copy(
                ub_hbm.at[u1d.at[sl]], ub_v.at[sl], sem_b))
            bias_copies.append(pltpu.async_copy(
                ib_hbm.at[i1d.at[sl]], ib_v.at[sl], sem_b))

        for h in range(2):
            row_copies = []
            for jj in range(n_chunks // 2):
                j = h * (n_chunks // 2) + jj
                src = pl.ds(j * _CHUNK, _CHUNK)
                dst = pl.ds(jj * _CHUNK, _CHUNK)
                row_copies.append(pltpu.async_copy(
                    uf_hbm.at[u2d.at[src]], urows_v.at[dst], sem_f))
                row_copies.append(pltpu.async_copy(
                    if_hbm.at[i2d.at[src]], irows_v.at[dst], sem_f))
            for c in row_copies:
                c.wait()
            if h == 0:
                for c in bias_copies:
                    c.wait()

            def group_body(g, _):
                p0 = h * (b_per_w // 2) + g * _LANES
                sl = pl.ds(p0, _LANES)
                rows = lax.iota(jnp.int32, _LANES) + g * _LANES
                ucol = (lax.shift_right_logical(u1d[sl], row_s)
                        & (_PACK - 1)) * d2
                icol = (lax.shift_right_logical(i1d[sl], row_s)
                        & (_PACK - 1)) * d2
                acc = ub_v[sl] + ib_v[sl]
                for dp in range(d2):
                    iu = plsc.load_gather(urows_v, [rows, ucol + dp])
                    iv = plsc.load_gather(irows_v, [rows, icol + dp])
                    ue, uo = plsc.unpack(
                        plsc.bitcast(iu, jnp.bfloat16),
                        format=plsc.PackFormat.INTERLEAVED)
                    ve, vo = plsc.unpack(
                        plsc.bitcast(iv, jnp.bfloat16),
                        format=plsc.PackFormat.INTERLEAVED)
                    acc = acc + ue * ve + uo * vo
                out_v[sl] = acc
                return 0

            lax.fori_loop(0, n_groups_half, group_body, 0)

        pltpu.sync_copy(out_v, out_hbm.at[pl.ds(base, b_per_w)])

    return k(idx_flat, uf_p, if_p, user_biases, item_biases)


def kernel(user_item_tuple, user_factors, item_factors, user_biases, item_biases):
    batch = user_item_tuple.shape[0]
    d = user_factors.shape[1]
    b_per_w = batch // _N_WORKERS
    idx_flat = jnp.concatenate(
        [user_item_tuple[:, 0].reshape(_N_WORKERS, b_per_w),
         user_item_tuple[:, 1].reshape(_N_WORKERS, b_per_w)],
        axis=1).reshape(-1)
    uf_p = _pack_table(user_factors.T)
    if_p = _pack_table(item_factors.T)
    return _mf_call(idx_flat, uf_p, if_p,
                    user_biases.reshape(-1), item_biases.reshape(-1),
                    b_per_w, d // 2)


# speedup vs baseline: 1.7003x; 1.1569x over previous
import functools

import jax
import jax.numpy as jnp
import numpy as np
from jax import lax
from jax.experimental import pallas as pl
from jax.experimental.pallas import tpu as pltpu
from jax.experimental.pallas import tpu_sc as plsc

_N_WORKERS = 32
_CHUNK = 128
_LANES = 16
_PACK = 8
_BLK = 32768
_ROWS = _BLK // _PACK


def _sel_const(d, parity):
    e = np.zeros((_PACK * d, _PACK * (d // 2)), np.float32)
    for c in range(_PACK):
        for k in range(d // 2):
            e[c * d + 2 * k + parity, (d // 2) * c + k] = 1.0
    return jnp.asarray(e, dtype=jnp.bfloat16)


def _pack_body(x_ref, ee_ref, eo_ref, o_ref):
    x = x_ref[...].astype(jnp.bfloat16)
    xs = jnp.concatenate(
        [x[:, c * _ROWS:(c + 1) * _ROWS] for c in range(_PACK)], axis=0)
    acc_e = lax.dot_general(xs, ee_ref[...], (((0,), (0,)), ((), ())),
                            preferred_element_type=jnp.float32)
    acc_o = lax.dot_general(xs, eo_ref[...], (((0,), (0,)), ((), ())),
                            preferred_element_type=jnp.float32)
    o_ref[...] = pltpu.pack_elementwise(
        [acc_e, acc_o], packed_dtype=jnp.bfloat16)


def _pack_table(table_t):
    d, n = table_t.shape
    grid = (n + _BLK - 1) // _BLK
    sel_shape = (_PACK * d, _PACK * (d // 2))
    return pl.pallas_call(
        _pack_body,
        grid=(grid,),
        in_specs=[
            pl.BlockSpec((d, _BLK), lambda g: (0, g)),
            pl.BlockSpec(sel_shape, lambda g: (0, 0)),
            pl.BlockSpec(sel_shape, lambda g: (0, 0)),
        ],
        out_specs=pl.BlockSpec((_ROWS, _PACK * (d // 2)), lambda g: (g, 0)),
        out_shape=jax.ShapeDtypeStruct(
            (grid * _ROWS, _PACK * (d // 2)), jnp.int32),
        compiler_params=pltpu.CompilerParams(
            dimension_semantics=("arbitrary",)),
    )(table_t, _sel_const(d, 0), _sel_const(d, 1))


@functools.partial(jax.jit, static_argnums=(5, 6))
def _mf_call(idx_flat, uf_p, if_p, user_biases, item_biases, b_per_w, d2):
    batch = idx_flat.shape[0] // 2
    n_chunks = b_per_w // _CHUNK
    n_groups_half = (b_per_w // 2) // _LANES
    mesh = plsc.VectorSubcoreMesh(core_axis_name="c", subcore_axis_name="s")

    @functools.partial(
        pl.kernel,
        out_type=jax.ShapeDtypeStruct((batch,), jnp.float32),
        mesh=mesh,
        compiler_params=pltpu.CompilerParams(
            needs_layout_passes=False, use_tc_tiling_on_sc=True),
        scratch_types=[
            pltpu.VMEM((b_per_w,), jnp.int32),
            pltpu.VMEM((b_per_w,), jnp.int32),
            pltpu.VMEM((b_per_w,), jnp.int32),
            pltpu.VMEM((b_per_w,), jnp.int32),
            pltpu.VMEM((b_per_w // 2, _PACK * d2), jnp.int32),
            pltpu.VMEM((b_per_w // 2, _PACK * d2), jnp.int32),
            pltpu.VMEM((b_per_w,), jnp.float32),
            pltpu.VMEM((b_per_w,), jnp.float32),
            pltpu.VMEM((b_per_w,), jnp.float32),
            pltpu.SemaphoreType.DMA,
            pltpu.SemaphoreType.DMA,
        ],
    )
    def k(idx_hbm, uf_hbm, if_hbm, ub_hbm, ib_hbm, out_hbm,
          u1d, i1d, u2d, i2d, urows_v, irows_v, ub_v, ib_v, out_v,
          sem_b, sem_f):
        wid = lax.axis_index("s") * 2 + lax.axis_index("c")
        base = wid * b_per_w
        pltpu.sync_copy(idx_hbm.at[pl.ds(2 * base, b_per_w)], u1d)
        pltpu.sync_copy(idx_hbm.at[pl.ds(2 * base + b_per_w, b_per_w)], i1d)

        blk_s = _BLK.bit_length() - 1
        row_s = _ROWS.bit_length() - 1

        def packed_row(u):
            return (lax.shift_left(lax.shift_right_logical(u, blk_s), row_s)
                    | (u & (_ROWS - 1)))

        def halve_body(l, _):
            sl = pl.ds(l * _LANES, _LANES)
            u2d[sl] = packed_row(u1d[sl])
            i2d[sl] = packed_row(i1d[sl])
            return 0

        lax.fori_loop(0, b_per_w // _LANES, halve_body, 0)

        bias_copies = []
        for j in range(n_chunks):
            sl = pl.ds(j * _CHUNK, _CHUNK)
            bias_copies.append(pltpu.async_copy(
                ub_hbm.at[u1d.at[sl]], ub_v.at[sl], sem_b))
            bias_copies.append(pltpu.async_copy(
                ib_hbm.at[i1d.at[sl]], ib_v.at[sl], sem_b))

        for h in range(2):
            row_copies = []
            for jj in range(n_chunks // 2):
                j = h * (n_chunks // 2) + jj
                src = pl.ds(j * _CHUNK, _CHUNK)
                dst = pl.ds(jj * _CHUNK, _CHUNK)
                row_copies.append(pltpu.async_copy(
                    uf_hbm.at[u2d.at[src]], urows_v.at[dst], sem_f))
                row_copies.append(pltpu.async_copy(
                    if_hbm.at[i2d.at[src]], irows_v.at[dst], sem_f))
            for c in row_copies:
                c.wait()
            if h == 0:
                for c in bias_copies:
                    c.wait()

            def group_body(g, _):
                p0 = h * (b_per_w // 2) + g * _LANES
                sl = pl.ds(p0, _LANES)
                rows = lax.iota(jnp.int32, _LANES) + g * _LANES
                ucol = (lax.shift_right_logical(u1d[sl], row_s)
                        & (_PACK - 1)) * d2
                icol = (lax.shift_right_logical(i1d[sl], row_s)
                        & (_PACK - 1)) * d2
                acc = ub_v[sl] + ib_v[sl]
                for dp in range(d2):
                    iu = plsc.load_gather(urows_v, [rows, ucol + dp])
                    iv = plsc.load_gather(irows_v, [rows, icol + dp])
                    ue, uo = plsc.unpack(
                        plsc.bitcast(iu, jnp.bfloat16),
                        format=plsc.PackFormat.INTERLEAVED)
                    ve, vo = plsc.unpack(
                        plsc.bitcast(iv, jnp.bfloat16),
                        format=plsc.PackFormat.INTERLEAVED)
                    acc = acc + ue * ve + uo * vo
                out_v[sl] = acc
                return 0

            lax.fori_loop(0, n_groups_half, group_body, 0)

        pltpu.sync_copy(out_v, out_hbm.at[pl.ds(base, b_per_w)])

    return k(idx_flat, uf_p, if_p, user_biases, item_biases)


def kernel(user_item_tuple, user_factors, item_factors, user_biases, item_biases):
    batch = user_item_tuple.shape[0]
    d = user_factors.shape[1]
    b_per_w = batch // _N_WORKERS
    idx_flat = jnp.concatenate(
        [user_item_tuple[:, 0].reshape(_N_WORKERS, b_per_w),
         user_item_tuple[:, 1].reshape(_N_WORKERS, b_per_w)],
        axis=1).reshape(-1)
    uf_p = _pack_table(user_factors.T)
    if_p = _pack_table(item_factors.T)
    return _mf_call(idx_flat, uf_p, if_p,
                    user_biases.reshape(-1), item_biases.reshape(-1),
                    b_per_w, d // 2)


# speedup vs baseline: 1.7250x; 1.0145x over previous
import functools

import jax
import jax.numpy as jnp
import numpy as np
from jax import lax
from jax.experimental import pallas as pl
from jax.experimental.pallas import tpu as pltpu
from jax.experimental.pallas import tpu_sc as plsc

_N_WORKERS = 32
_CHUNK = 128
_LANES = 16
_PACK = 8
_BLK = 65536
_ROWS = _BLK // _PACK


def _sel_const(d, parity):
    e = np.zeros((_PACK * d, _PACK * (d // 2)), np.float32)
    for c in range(_PACK):
        for k in range(d // 2):
            e[c * d + 2 * k + parity, (d // 2) * c + k] = 1.0
    return jnp.asarray(e, dtype=jnp.bfloat16)


def _pack_body(x_ref, ee_ref, eo_ref, o_ref):
    x = x_ref[...].astype(jnp.bfloat16)
    xs = jnp.concatenate(
        [x[:, c * _ROWS:(c + 1) * _ROWS] for c in range(_PACK)], axis=0)
    acc_e = lax.dot_general(xs, ee_ref[...], (((0,), (0,)), ((), ())),
                            preferred_element_type=jnp.float32)
    acc_o = lax.dot_general(xs, eo_ref[...], (((0,), (0,)), ((), ())),
                            preferred_element_type=jnp.float32)
    o_ref[...] = pltpu.pack_elementwise(
        [acc_e, acc_o], packed_dtype=jnp.bfloat16)


def _pack_table(table_t):
    d, n = table_t.shape
    grid = (n + _BLK - 1) // _BLK
    sel_shape = (_PACK * d, _PACK * (d // 2))
    return pl.pallas_call(
        _pack_body,
        grid=(grid,),
        in_specs=[
            pl.BlockSpec((d, _BLK), lambda g: (0, g)),
            pl.BlockSpec(sel_shape, lambda g: (0, 0)),
            pl.BlockSpec(sel_shape, lambda g: (0, 0)),
        ],
        out_specs=pl.BlockSpec((_ROWS, _PACK * (d // 2)), lambda g: (g, 0)),
        out_shape=jax.ShapeDtypeStruct(
            (grid * _ROWS, _PACK * (d // 2)), jnp.int32),
        compiler_params=pltpu.CompilerParams(
            dimension_semantics=("arbitrary",)),
    )(table_t, _sel_const(d, 0), _sel_const(d, 1))


@functools.partial(jax.jit, static_argnums=(5, 6))
def _mf_call(idx_flat, uf_p, if_p, user_biases, item_biases, b_per_w, d2):
    batch = idx_flat.shape[0] // 2
    n_chunks = b_per_w // _CHUNK
    n_groups_half = (b_per_w // 2) // _LANES
    mesh = plsc.VectorSubcoreMesh(core_axis_name="c", subcore_axis_name="s")

    @functools.partial(
        pl.kernel,
        out_type=jax.ShapeDtypeStruct((batch,), jnp.float32),
        mesh=mesh,
        compiler_params=pltpu.CompilerParams(
            needs_layout_passes=False, use_tc_tiling_on_sc=True),
        scratch_types=[
            pltpu.VMEM((b_per_w,), jnp.int32),
            pltpu.VMEM((b_per_w,), jnp.int32),
            pltpu.VMEM((b_per_w,), jnp.int32),
            pltpu.VMEM((b_per_w,), jnp.int32),
            pltpu.VMEM((b_per_w // 2, _PACK * d2), jnp.int32),
            pltpu.VMEM((b_per_w // 2, _PACK * d2), jnp.int32),
            pltpu.VMEM((b_per_w,), jnp.float32),
            pltpu.VMEM((b_per_w,), jnp.float32),
            pltpu.VMEM((b_per_w,), jnp.float32),
            pltpu.SemaphoreType.DMA,
            pltpu.SemaphoreType.DMA,
        ],
    )
    def k(idx_hbm, uf_hbm, if_hbm, ub_hbm, ib_hbm, out_hbm,
          u1d, i1d, u2d, i2d, urows_v, irows_v, ub_v, ib_v, out_v,
          sem_b, sem_f):
        wid = lax.axis_index("s") * 2 + lax.axis_index("c")
        base = wid * b_per_w
        pltpu.sync_copy(idx_hbm.at[pl.ds(2 * base, b_per_w)], u1d)
        pltpu.sync_copy(idx_hbm.at[pl.ds(2 * base + b_per_w, b_per_w)], i1d)

        blk_s = _BLK.bit_length() - 1
        row_s = _ROWS.bit_length() - 1

        def packed_row(u):
            return (lax.shift_left(lax.shift_right_logical(u, blk_s), row_s)
                    | (u & (_ROWS - 1)))

        def halve_body(l, _):
            sl = pl.ds(l * _LANES, _LANES)
            u2d[sl] = packed_row(u1d[sl])
            i2d[sl] = packed_row(i1d[sl])
            return 0

        lax.fori_loop(0, b_per_w // _LANES, halve_body, 0)

        bias_copies = []
        for j in range(n_chunks):
            sl = pl.ds(j * _CHUNK, _CHUNK)
            bias_copies.append(pltpu.async_copy(
                ub_hbm.at[u1d.at[sl]], ub_v.at[sl], sem_b))
            bias_copies.append(pltpu.async_copy(
                ib_hbm.at[i1d.at[sl]], ib_v.at[sl], sem_b))

        for h in range(2):
            row_copies = []
            for jj in range(n_chunks // 2):
                j = h * (n_chunks // 2) + jj
                src = pl.ds(j * _CHUNK, _CHUNK)
                dst = pl.ds(jj * _CHUNK, _CHUNK)
                row_copies.append(pltpu.async_copy(
                    uf_hbm.at[u2d.at[src]], urows_v.at[dst], sem_f))
                row_copies.append(pltpu.async_copy(
                    if_hbm.at[i2d.at[src]], irows_v.at[dst], sem_f))
            for c in row_copies:
                c.wait()
            if h == 0:
                for c in bias_copies:
                    c.wait()

            def group_body(g, _):
                p0 = h * (b_per_w // 2) + g * _LANES
                sl = pl.ds(p0, _LANES)
                rows = lax.iota(jnp.int32, _LANES) + g * _LANES
                ucol = (lax.shift_right_logical(u1d[sl], row_s)
                        & (_PACK - 1)) * d2
                icol = (lax.shift_right_logical(i1d[sl], row_s)
                        & (_PACK - 1)) * d2
                acc = ub_v[sl] + ib_v[sl]
                for dp in range(d2):
                    iu = plsc.load_gather(urows_v, [rows, ucol + dp])
                    iv = plsc.load_gather(irows_v, [rows, icol + dp])
                    ue, uo = plsc.unpack(
                        plsc.bitcast(iu, jnp.bfloat16),
                        format=plsc.PackFormat.INTERLEAVED)
                    ve, vo = plsc.unpack(
                        plsc.bitcast(iv, jnp.bfloat16),
                        format=plsc.PackFormat.INTERLEAVED)
                    acc = acc + ue * ve + uo * vo
                out_v[sl] = acc
                return 0

            lax.fori_loop(0, n_groups_half, group_body, 0)

        pltpu.sync_copy(out_v, out_hbm.at[pl.ds(base, b_per_w)])

    return k(idx_flat, uf_p, if_p, user_biases, item_biases)


def kernel(user_item_tuple, user_factors, item_factors, user_biases, item_biases):
    batch = user_item_tuple.shape[0]
    d = user_factors.shape[1]
    b_per_w = batch // _N_WORKERS
    idx_flat = jnp.concatenate(
        [user_item_tuple[:, 0].reshape(_N_WORKERS, b_per_w),
         user_item_tuple[:, 1].reshape(_N_WORKERS, b_per_w)],
        axis=1).reshape(-1)
    uf_p = _pack_table(user_factors.T)
    if_p = _pack_table(item_factors.T)
    return _mf_call(idx_flat, uf_p, if_p,
                    user_biases.reshape(-1), item_biases.reshape(-1),
                    b_per_w, d // 2)
